# Initial kernel scaffold; baseline (speedup 1.0000x reference)
import jax
import jax.numpy as jnp
from jax.experimental import pallas as pl


def kernel(input_mids, input_wids, input_cnt, mc_table, xl_table, xr_table):
    raise NotImplementedError("write your pallas kernel here")


# speedup vs baseline: 1.4635x; 1.4635x over previous
import jax
import jax.numpy as jnp
from jax import lax
from jax.experimental import pallas as pl
from jax.experimental.pallas import tpu as pltpu
from jax.experimental.pallas import tpu_sc as plsc

B, L, VOCAB, STYLE = 16, 2048, 100000, 18
EPS = 1e-7

HALF = 1024
STAGE = 1040
IDXN = 2 * STAGE
GB = 104
NBLK = IDXN // GB
NG = HALF // 16
PAD = STAGE + 112
VROWS = (VOCAB * STYLE) // 16


def _sc_body(wids_hbm, mids_hbm, cnt_hbm, mc_hbm, xl_hbm, xr_hbm,
             part_hbm, head_hbm,
             wids_v, mids_v, cnt_v, mc_v, headidx_v, headidx2, idxbuf,
             xlbuf, xrbuf, headrows,
             row_ref, head_pad,
             sem_l, sem_r, sem_h):
    k = lax.axis_index("c")
    b = lax.axis_index("s")
    start = k * HALF
    base = b * L + start

    pltpu.sync_copy(wids_hbm.at[pl.ds(base, STAGE)], wids_v)
    pltpu.sync_copy(mids_hbm.at[pl.ds(base, STAGE)], mids_v)
    pltpu.sync_copy(cnt_hbm, cnt_v)
    pltpu.sync_copy(mc_hbm, mc_v)
    pltpu.sync_copy(wids_hbm.at[pl.ds(b * L, 16)], headidx_v)

    iota = lax.iota(jnp.int32, 16)
    zeros16 = jnp.zeros((16,), jnp.int32)

    def build(g, carry):
        l = g * 16 + iota
        w = plsc.load_gather(wids_v, [l])
        r0 = (w * STYLE) >> 4
        plsc.store_scatter(idxbuf, [l * 2], r0)
        plsc.store_scatter(idxbuf, [l * 2 + 1], r0 + 1)
        return carry

    lax.fori_loop(0, STAGE // 16, build, jnp.int32(0))

    hw_all = plsc.load_gather(headidx_v, [iota])
    w0 = jnp.sum(jnp.where(iota == 0, hw_all, 0))
    hw = jnp.full((16,), w0, jnp.int32)
    hr0 = (hw * STYLE) >> 4
    off0 = (hw * STYLE) & 15
    plsc.store_scatter(headidx2, [iota],
                       jnp.where(iota == 1, hr0 + 1, hr0))

    pltpu.async_copy(xl_hbm.at[headidx2], headrows, sem_h).wait()
    copies = []
    for blk in range(NBLK):
        idx = idxbuf.at[pl.ds(blk * GB, GB)]
        copies.append(pltpu.async_copy(
            xl_hbm.at[idx], xlbuf.at[pl.ds(blk * GB, GB)], sem_l))
        copies.append(pltpu.async_copy(
            xr_hbm.at[idx], xrbuf.at[pl.ds(blk * GB, GB)], sem_r))
    for c_ in copies:
        c_.wait()

    cv = plsc.load_gather(cnt_v, [jnp.full((16,), b, jnp.int32)])

    def head_at(t):
        return plsc.load_gather(headrows, [t >> 4, t & 15])

    hd = [head_at(off0 + d) for d in range(STYLE)]

    zf = jnp.zeros((16,), jnp.float32)
    init = tuple([zf] * (2 * STYLE))

    def body(g, acc):
        ri = g * 16 + iota
        rn = ri + 1
        w_i = plsc.load_gather(wids_v, [ri])
        w_n = plsc.load_gather(wids_v, [rn])
        mids_i = plsc.load_gather(mids_v, [ri])
        mids_n = plsc.load_gather(mids_v, [rn])
        off_i = (w_i * STYLE) & 15
        off_n = (w_n * STYLE) & 15
        bri = ri * 2
        brn = rn * 2
        pos = ri + start
        mv = (pos >= 1) & (pos < cv - 1)
        mh = (pos >= 1) & (pos < cv)
        acc = list(acc)
        for d in range(STYLE):
            t_i = off_i + d
            t_n = off_n + d
            rri = bri + (t_i >> 4)
            cci = t_i & 15
            rrn = brn + (t_n >> 4)
            ccn = t_n & 15
            xl_i = plsc.load_gather(xlbuf, [rri, cci])
            xr_i = plsc.load_gather(xrbuf, [rri, cci])
            xl_n = plsc.load_gather(xlbuf, [rrn, ccn])
            mb_i = plsc.load_gather(mc_v, [mids_i * STYLE + d])
            mb_n = plsc.load_gather(mc_v, [mids_n * STYLE + d])
            dv = mb_i * xr_i - mb_n * xl_n
            acc[d] = acc[d] + jnp.where(mv, dv * dv, 0.0)
            dh = mb_i * (0.5 * (xl_i + xr_i) - hd[d])
            acc[STYLE + d] = acc[STYLE + d] + jnp.where(mh, dh * dh, 0.0)
        return tuple(acc)

    acc = lax.fori_loop(0, NG, body, init)

    for w in range(2):
        lo = jnp.zeros((16,), jnp.float32)
        hi = jnp.zeros((16,), jnp.float32)
        for d in range(STYLE):
            s = jnp.sum(acc[w * STYLE + d])
            sp = jnp.full((16,), s, jnp.float32)
            if d < 16:
                lo = jnp.where(iota == d, sp, lo)
            else:
                hi = jnp.where(iota == (d - 16), sp, hi)
        plsc.store_scatter(row_ref, [iota + 32 * w], lo)
        plsc.store_scatter(row_ref, [iota + 32 * w + 16], hi)
    r = k * B + b
    pltpu.sync_copy(row_ref, part_hbm.at[r])

    @pl.when(k == 0)
    def _():
        plsc.store_scatter(head_pad, [iota], head_at(off0 + iota))
        plsc.store_scatter(head_pad, [iota + 16],
                           head_at(off0 + jnp.minimum(iota + 16, STYLE - 1)))
        pltpu.sync_copy(head_pad, head_hbm.at[b])


@jax.jit
def _sc_call(wids_flat, mids_flat, cnt, mc_flat, xl_table, xr_table):
    xl16 = xl_table.reshape(VROWS, 16)
    xr16 = xr_table.reshape(VROWS, 16)
    mesh = plsc.VectorSubcoreMesh(core_axis_name="c", subcore_axis_name="s",
                                  num_cores=2, num_subcores=16)
    f = pl.kernel(
        _sc_body,
        out_type=(
            jax.ShapeDtypeStruct((2 * B, 64), jnp.float32),
            jax.ShapeDtypeStruct((B, 32), jnp.float32),
        ),
        mesh=mesh,
        scratch_types=[
            pltpu.VMEM((STAGE,), jnp.int32),
            pltpu.VMEM((STAGE,), jnp.int32),
            pltpu.VMEM((16,), jnp.int32),
            pltpu.VMEM((512,), jnp.float32),
            pltpu.VMEM((16,), jnp.int32),
            pltpu.VMEM((16,), jnp.int32),
            pltpu.VMEM((IDXN,), jnp.int32),
            pltpu.VMEM((IDXN, 16), jnp.float32),
            pltpu.VMEM((IDXN, 16), jnp.float32),
            pltpu.VMEM((16, 16), jnp.float32),
            pltpu.VMEM((64,), jnp.float32),
            pltpu.VMEM((32,), jnp.float32),
            pltpu.SemaphoreType.DMA,
            pltpu.SemaphoreType.DMA,
            pltpu.SemaphoreType.DMA,
        ],
        compiler_params=pltpu.CompilerParams(
            needs_layout_passes=False, use_tc_tiling_on_sc=False),
    )
    return f(wids_flat, mids_flat, cnt, mc_flat, xl16, xr16)


def kernel(input_mids, input_wids, input_cnt, mc_table, xl_table, xr_table):
    zpad = jnp.zeros((PAD,), jnp.int32)
    wids_flat = jnp.concatenate([input_wids.reshape(-1).astype(jnp.int32), zpad])
    mids_flat = jnp.concatenate([input_mids.reshape(-1).astype(jnp.int32), zpad])
    mc_flat = jnp.zeros((512,), jnp.float32).at[:STYLE * STYLE].set(
        mc_table.reshape(-1))
    part, head_p = _sc_call(wids_flat, mids_flat, input_cnt.astype(jnp.int32),
                            mc_flat, xl_table, xr_table)
    sums = part[:B] + part[B:]
    sv = sums[:, 0:STYLE]
    sh = sums[:, 32:32 + STYLE]
    cnt_f = input_cnt.astype(jnp.float32)[:, None]
    out = 0.5 * jnp.sqrt(sv / (cnt_f + EPS)) + 0.5 * jnp.sqrt(sh / (cnt_f + EPS))
    head = head_p[:, :STYLE]
    return (out, head)
